# baseline (speedup 1.0000x reference)
import functools

import jax
import jax.numpy as jnp
from jax import lax
from jax.experimental import pallas as pl
from jax.experimental.pallas import tpu as pltpu
from jax.experimental.pallas import tpu_sc as plsc

_B, _N, _K, _D = 8, 1024, 32, 3
_C = 30
_CP = 32
_M = _K * _CP
_NCLS = 40

_NC, _NS = 2, 16
_NW = _NC * _NS
_ROWS = _D * _B * _N
_TOT = _D * _B * _N * _K
_PER_W = _TOT // _NW
_CHUNK = 2048
_STEPS = _PER_W // _CHUNK

_mesh = plsc.VectorSubcoreMesh(core_axis_name="c", subcore_axis_name="s")


@functools.partial(
    pl.kernel,
    mesh=_mesh,
    compiler_params=pltpu.CompilerParams(use_tc_tiling_on_sc=False),
    out_type=jax.ShapeDtypeStruct((_TOT, _CP), jnp.bfloat16),
    scratch_types=[
        pltpu.VMEM((_PER_W,), jnp.int32),
        pltpu.VMEM((_CHUNK, _CP), jnp.bfloat16),
        pltpu.SemaphoreType.DMA,
    ],
)
def _sc_gather(table_hbm, idx_hbm, out_hbm, idx_v, rows_v, sem):
    wid = lax.axis_index("s") * _NC + lax.axis_index("c")
    base = wid * _PER_W
    pltpu.sync_copy(idx_hbm.at[pl.ds(base, _PER_W)], idx_v)
    for i in range(_STEPS):
        pltpu.async_copy(
            table_hbm.at[idx_v.at[pl.ds(i * _CHUNK, _CHUNK)]], rows_v, sem
        ).wait()
        pltpu.sync_copy(rows_v, out_hbm.at[pl.ds(base + i * _CHUNK, _CHUNK)])


def _layer_body(w_ref, g_ref, h16_ref, h32_ref):
    w = w_ref[...]
    m = jnp.max(w, axis=0, keepdims=True)
    e = jnp.exp(w - m)
    wsm = (e / jnp.sum(e, axis=0, keepdims=True)).astype(jnp.bfloat16)
    mm = [
        lax.dot_general(
            g_ref[dd, 0], wsm, (((1,), (0,)), ((), ())),
            preferred_element_type=jnp.float32,
        )
        for dd in range(_D)
    ]
    n1 = jnp.sqrt(mm[0] * mm[0] + mm[1] * mm[1] + mm[2] * mm[2])
    u = [mm[dd] / (n1 + 1e-8) for dd in range(_D)]
    n2 = jnp.sqrt(u[0] * u[0] + u[1] * u[1] + u[2] * u[2])
    scale = jnp.tanh(n2) / (n2 + 1e-8)
    for dd in range(_D):
        h = u[dd] * scale
        h16_ref[dd, 0] = h.astype(jnp.bfloat16)
        h32_ref[dd, 0] = h


def _layer_tc(warr, g):
    return pl.pallas_call(
        _layer_body,
        grid=(_B,),
        in_specs=[
            pl.BlockSpec((_M, _CP), lambda b: (0, 0)),
            pl.BlockSpec((_D, 1, _N, _M), lambda b: (0, b, 0, 0)),
        ],
        out_specs=[pl.BlockSpec((_D, 1, _N, _CP), lambda b: (0, b, 0, 0)),
                   pl.BlockSpec((_D, 1, _N, _CP), lambda b: (0, b, 0, 0))],
        out_shape=[jax.ShapeDtypeStruct((_D, _B, _N, _CP), jnp.bfloat16),
                   jax.ShapeDtypeStruct((_D, _B, _N, _CP), jnp.float32)],
    )(warr, g)


def _acos(t):
    ax = jnp.abs(t)
    p = 1.5707288 + ax * (-0.2121144 + ax * (0.0742610 + ax * (-0.0187293)))
    r = jnp.sqrt(jnp.maximum(1.0 - ax, 0.0)) * p
    return jnp.where(t >= 0.0, r, 3.14159265358979 - r)


def _bf(v):
    return v.astype(jnp.bfloat16).astype(jnp.float32)


def _head_body(h_ref, wl_ref, bl_ref, out_ref):
    for b in range(_B):
        hb = [h_ref[dd, b] for dd in range(_D)]
        m = [jnp.mean(v, axis=0, keepdims=True) for v in hb]
        md = jnp.sqrt(m[0] * m[0] + m[1] * m[1] + m[2] * m[2]) + 1e-8
        xd = jnp.sqrt(hb[0] * hb[0] + hb[1] * hb[1] + hb[2] * hb[2]) + 1e-8
        dots = (_bf(hb[0] / xd) * _bf(m[0] / md)
                + _bf(hb[1] / xd) * _bf(m[1] / md)
                + _bf(hb[2] / xd) * _bf(m[2] / md))
        dist = _acos(jnp.clip(dots, -0.999, 0.999))
        feat = jnp.mean(dist, axis=0, keepdims=True)
        lg = lax.dot_general(
            feat.astype(jnp.bfloat16), wl_ref[...].astype(jnp.bfloat16),
            (((1,), (0,)), ((), ())),
            preferred_element_type=jnp.float32,
        )
        out_ref[pl.ds(b, 1), :] = lg + bl_ref[...]


def _head(h, wl_t, bl_row):
    return pl.pallas_call(
        _head_body,
        out_shape=jax.ShapeDtypeStruct((_B, _NCLS), jnp.float32),
    )(h, wl_t, bl_row)


def _arrange(W, C):
    Wr = W.astype(jnp.float32).reshape(_C, _K, C)
    Wp = jnp.pad(Wr, ((0, _CP - _C), (0, 0), (0, _CP - C)),
                 constant_values=-1e30)
    return Wp.reshape(_CP, _M).T


def kernel(x, neighborhood_matrix, W1, W2, W3, W4, W5, Wl, bl):
    xt = jnp.transpose(x.astype(jnp.float32)[:, :, 0, :], (2, 0, 1))
    h = jnp.pad(xt[..., None], ((0, 0), (0, 0), (0, 0), (0, _CP - 1)))

    offs = (jnp.arange(_D, dtype=jnp.int32)[:, None, None, None] * _B
            + jnp.arange(_B, dtype=jnp.int32)[None, :, None, None]) * _N
    idx = (neighborhood_matrix.astype(jnp.int32)[None] + offs).reshape(_TOT)

    wl_t = jnp.pad(Wl.astype(jnp.float32), ((0, 0), (0, _CP - _C))).T
    bl_row = bl.astype(jnp.float32).reshape(1, _NCLS)

    ws = (_arrange(W1, 1), _arrange(W2, _C), _arrange(W3, _C),
          _arrange(W4, _C), _arrange(W5, _C))
    table = h.astype(jnp.bfloat16)
    for warr in ws:
        g = _sc_gather(table.reshape(_ROWS, _CP), idx)
        table, h32 = _layer_tc(warr, g.reshape(_D, _B, _N, _M))
    return _head(h32, wl_t, bl_row)


# speedup vs baseline: 1.3533x; 1.3533x over previous
import functools

import jax
import jax.numpy as jnp
from jax import lax
from jax.experimental import pallas as pl
from jax.experimental.pallas import tpu as pltpu
from jax.experimental.pallas import tpu_sc as plsc

_B, _N, _K, _D = 8, 1024, 32, 3
_C = 30
_CP = 32
_M = _K * _CP
_NCLS = 40

_NC, _NS = 2, 16
_NW = _NC * _NS
_ROWS = _D * _B * _N
_TOT = _D * _B * _N * _K
_PER_W = _TOT // _NW
_CHUNK = 1024
_STEPS = _PER_W // _CHUNK

_mesh = plsc.VectorSubcoreMesh(core_axis_name="c", subcore_axis_name="s")


@functools.partial(
    pl.kernel,
    mesh=_mesh,
    compiler_params=pltpu.CompilerParams(use_tc_tiling_on_sc=False),
    out_type=jax.ShapeDtypeStruct((_TOT, _CP), jnp.float32),
    scratch_types=[
        pltpu.VMEM((_PER_W,), jnp.int32),
        pltpu.VMEM((_CHUNK, _CP), jnp.float32),
        pltpu.VMEM((_CHUNK, _CP), jnp.float32),
        pltpu.SemaphoreType.DMA,
        pltpu.SemaphoreType.DMA,
    ],
)
def _sc_gather(table_hbm, idx_hbm, out_hbm, idx_v, rows_a, rows_b, sem_a,
               sem_b):
    wid = lax.axis_index("s") * _NC + lax.axis_index("c")
    base = wid * _PER_W
    pltpu.sync_copy(idx_hbm.at[pl.ds(base, _PER_W)], idx_v)
    bufs = ((rows_a, sem_a), (rows_b, sem_b))

    def start(i):
        rows, sem = bufs[i % 2]
        return pltpu.async_copy(
            table_hbm.at[idx_v.at[pl.ds(i * _CHUNK, _CHUNK)]], rows, sem)

    pending = start(0)
    for i in range(_STEPS):
        nxt = start(i + 1) if i + 1 < _STEPS else None
        pending.wait()
        pltpu.sync_copy(bufs[i % 2][0],
                        out_hbm.at[pl.ds(base + i * _CHUNK, _CHUNK)])
        pending = nxt


def _layer_body(w_ref, g_ref, h_ref):
    w = w_ref[...]
    m = jnp.max(w, axis=0, keepdims=True)
    e = jnp.exp(w - m)
    wsm = (e / jnp.sum(e, axis=0, keepdims=True)).astype(jnp.bfloat16)
    mm = [
        lax.dot_general(
            g_ref[dd, 0].astype(jnp.bfloat16), wsm, (((1,), (0,)), ((), ())),
            preferred_element_type=jnp.float32,
        )
        for dd in range(_D)
    ]
    n1 = jnp.sqrt(mm[0] * mm[0] + mm[1] * mm[1] + mm[2] * mm[2])
    u = [mm[dd] / (n1 + 1e-8) for dd in range(_D)]
    n2 = jnp.sqrt(u[0] * u[0] + u[1] * u[1] + u[2] * u[2])
    scale = jnp.tanh(n2) / (n2 + 1e-8)
    for dd in range(_D):
        h_ref[dd, 0] = u[dd] * scale


def _layer_tc(warr, g):
    return pl.pallas_call(
        _layer_body,
        grid=(_B,),
        in_specs=[
            pl.BlockSpec((_M, _CP), lambda b: (0, 0)),
            pl.BlockSpec((_D, 1, _N, _M), lambda b: (0, b, 0, 0)),
        ],
        out_specs=pl.BlockSpec((_D, 1, _N, _CP), lambda b: (0, b, 0, 0)),
        out_shape=jax.ShapeDtypeStruct((_D, _B, _N, _CP), jnp.float32),
    )(warr, g)


def _acos(t):
    ax = jnp.abs(t)
    p = 1.5707288 + ax * (-0.2121144 + ax * (0.0742610 + ax * (-0.0187293)))
    r = jnp.sqrt(jnp.maximum(1.0 - ax, 0.0)) * p
    return jnp.where(t >= 0.0, r, 3.14159265358979 - r)


def _bf(v):
    return v.astype(jnp.bfloat16).astype(jnp.float32)


def _head_body(h_ref, wl_ref, bl_ref, out_ref):
    for b in range(_B):
        hb = [h_ref[dd, b] for dd in range(_D)]
        m = [jnp.mean(v, axis=0, keepdims=True) for v in hb]
        md = jnp.sqrt(m[0] * m[0] + m[1] * m[1] + m[2] * m[2]) + 1e-8
        xd = jnp.sqrt(hb[0] * hb[0] + hb[1] * hb[1] + hb[2] * hb[2]) + 1e-8
        dots = (_bf(hb[0] / xd) * _bf(m[0] / md)
                + _bf(hb[1] / xd) * _bf(m[1] / md)
                + _bf(hb[2] / xd) * _bf(m[2] / md))
        dist = _acos(jnp.clip(dots, -0.999, 0.999))
        feat = jnp.mean(dist, axis=0, keepdims=True)
        lg = lax.dot_general(
            feat.astype(jnp.bfloat16), wl_ref[...].astype(jnp.bfloat16),
            (((1,), (0,)), ((), ())),
            preferred_element_type=jnp.float32,
        )
        out_ref[pl.ds(b, 1), :] = lg + bl_ref[...]


def _head(h, wl_t, bl_row):
    return pl.pallas_call(
        _head_body,
        out_shape=jax.ShapeDtypeStruct((_B, _NCLS), jnp.float32),
    )(h, wl_t, bl_row)


def _arrange(W, C):
    Wr = W.astype(jnp.float32).reshape(_C, _K, C)
    Wp = jnp.pad(Wr, ((0, _CP - _C), (0, 0), (0, _CP - C)),
                 constant_values=-1e30)
    return Wp.reshape(_CP, _M).T


def kernel(x, neighborhood_matrix, W1, W2, W3, W4, W5, Wl, bl):
    xt = jnp.transpose(x.astype(jnp.float32)[:, :, 0, :], (2, 0, 1))
    h = jnp.pad(xt[..., None], ((0, 0), (0, 0), (0, 0), (0, _CP - 1)))

    offs = (jnp.arange(_D, dtype=jnp.int32)[:, None, None, None] * _B
            + jnp.arange(_B, dtype=jnp.int32)[None, :, None, None]) * _N
    idx = (neighborhood_matrix.astype(jnp.int32)[None] + offs).reshape(_TOT)

    wl_t = jnp.pad(Wl.astype(jnp.float32), ((0, 0), (0, _CP - _C))).T
    bl_row = bl.astype(jnp.float32).reshape(1, _NCLS)

    ws = (_arrange(W1, 1), _arrange(W2, _C), _arrange(W3, _C),
          _arrange(W4, _C), _arrange(W5, _C))
    for warr in ws:
        g = _sc_gather(h.reshape(_ROWS, _CP), idx)
        h = _layer_tc(warr, g.reshape(_D, _B, _N, _M))
    return _head(h, wl_t, bl_row)
